# baseline (speedup 1.0000x reference)
import functools

import jax
import jax.numpy as jnp
from jax import lax
from jax.experimental import pallas as pl
from jax.experimental.pallas import tpu as pltpu
from jax.experimental.pallas import tpu_sc as plsc

N = 10000
E = 320000
D_FEAT = 128
HIDDEN = 64
N_GRAPHS = 64

NC = 2
NS = 16
NW = NC * NS
EPW = E // NW
CHUNK = 80
NCHUNK = EPW // CHUNK
NCH_RUN = 128
NCH_ALLOC = 130
EPW_PAD = NCH_ALLOC * CHUNK
N_PAD = 10240
RPT = N_PAD // NS
W = 128


def _sc_mesh():
    return plsc.VectorSubcoreMesh(
        core_axis_name="c", subcore_axis_name="s",
        num_cores=NC, num_subcores=NS)


def _deg_partial(dst3, ones, zeros):
    @functools.partial(
        pl.kernel,
        out_type=jax.ShapeDtypeStruct((NC, N_PAD, W), jnp.float32),
        mesh=_sc_mesh(),
        scratch_types=[
            pltpu.VMEM((NCH_ALLOC, CHUNK), jnp.int32),
            pltpu.VMEM((CHUNK, W), jnp.float32),
            pltpu.VMEM_SHARED((N_PAD, W), jnp.float32),
            pltpu.SemaphoreType.DMA,
        ],
    )
    def kern(dst_hbm, ones_hbm, zeros_hbm, out_hbm, dst_v, ones_v, acc_sh,
             dsem):
        c = lax.axis_index("c")
        s = lax.axis_index("s")
        wid = c * NS + s
        r0 = s * RPT
        pltpu.sync_copy(zeros_hbm.at[pl.ds(r0, RPT)], acc_sh.at[pl.ds(r0, RPT)])
        pltpu.sync_copy(ones_hbm, ones_v)
        pltpu.sync_copy(dst_hbm.at[wid], dst_v)
        plsc.subcore_barrier()

        def issue(j, carry):
            pltpu.async_copy(ones_v, acc_sh.at[dst_v.at[j]], dsem, add=True)
            return carry

        lax.fori_loop(0, NCHUNK, issue, 0)

        def drain(j, carry):
            pltpu.make_async_copy(ones_v, acc_sh.at[dst_v.at[j]], dsem).wait()
            return carry

        lax.fori_loop(0, NCHUNK, drain, 0)
        plsc.subcore_barrier()
        pltpu.sync_copy(acc_sh.at[pl.ds(r0, RPT)],
                        out_hbm.at[c, pl.ds(r0, RPT)])

    return kern(dst3, ones, zeros)


def _prop_partial(hs, src3, dst3, zeros):
    @functools.partial(
        pl.kernel,
        out_type=jax.ShapeDtypeStruct((NC, N_PAD, W), jnp.float32),
        mesh=_sc_mesh(),
        scratch_types=[
            pltpu.VMEM((EPW_PAD,), jnp.int32),
            pltpu.VMEM((NCH_ALLOC, CHUNK), jnp.int32),
            pltpu.VMEM((CHUNK, W), jnp.float32),
            pltpu.VMEM((CHUNK, W), jnp.float32),
            pltpu.VMEM_SHARED((N_PAD, W), jnp.float32),
            pltpu.SemaphoreType.DMA,
            pltpu.SemaphoreType.DMA,
            pltpu.SemaphoreType.DMA,
            pltpu.SemaphoreType.DMA,
        ],
    )
    def kern(hs_hbm, src_hbm, dst_hbm, zeros_hbm, out_hbm,
             src_v, dst_v, rows_a, rows_b, acc_sh,
             ga, gb, sa, sb):
        c = lax.axis_index("c")
        s = lax.axis_index("s")
        wid = c * NS + s
        r0 = s * RPT
        pltpu.sync_copy(zeros_hbm.at[pl.ds(r0, RPT)], acc_sh.at[pl.ds(r0, RPT)])
        pltpu.sync_copy(src_hbm.at[wid], src_v)
        pltpu.sync_copy(dst_hbm.at[wid], dst_v)
        plsc.subcore_barrier()

        def start_g(j, buf, sem):
            pltpu.async_copy(hs_hbm.at[src_v.at[pl.ds(j * CHUNK, CHUNK)]],
                             buf, sem)

        def wait_g(j, buf, sem):
            pltpu.make_async_copy(hs_hbm.at[src_v.at[pl.ds(j * CHUNK, CHUNK)]],
                                  buf, sem).wait()

        def start_s(j, buf, sem):
            pltpu.async_copy(buf, acc_sh.at[dst_v.at[j]], sem, add=True)

        def wait_s(j, buf, sem):
            pltpu.make_async_copy(buf, acc_sh.at[dst_v.at[j]], sem).wait()

        start_g(0, rows_a, ga)
        start_g(1, rows_b, gb)

        def body(t, carry):
            j0 = 2 * t
            wait_g(j0, rows_a, ga)
            start_s(j0, rows_a, sa)
            wait_g(j0 + 1, rows_b, gb)
            start_s(j0 + 1, rows_b, sb)
            wait_s(j0, rows_a, sa)
            start_g(j0 + 2, rows_a, ga)
            wait_s(j0 + 1, rows_b, sb)
            start_g(j0 + 3, rows_b, gb)
            return carry

        lax.fori_loop(0, NCH_RUN // 2, body, 0)
        wait_g(NCH_RUN, rows_a, ga)
        wait_g(NCH_RUN + 1, rows_b, gb)
        plsc.subcore_barrier()
        pltpu.sync_copy(acc_sh.at[pl.ds(r0, RPT)],
                        out_hbm.at[c, pl.ds(r0, RPT)])

    return kern(hs, src3, dst3, zeros)


def _scale1(x, W1, degp):
    def body(x_ref, w_ref, degp_ref, hs_ref, rns_ref):
        p = degp_ref[...]
        deg = p[0, :N, 0:1] + p[1, :N, 0:1]
        rns = lax.rsqrt(jnp.maximum(deg, 1.0))
        rns_ref[...] = rns
        h = jnp.dot(x_ref[...], w_ref[...],
                    preferred_element_type=jnp.float32) * rns
        hs_ref[:, :HIDDEN] = h
        hs_ref[:, HIDDEN:] = jnp.zeros((N, W - HIDDEN), jnp.float32)

    return pl.pallas_call(
        body,
        out_shape=(jax.ShapeDtypeStruct((N, W), jnp.float32),
                   jax.ShapeDtypeStruct((N, 1), jnp.float32)),
    )(x, W1, degp)


def _mid(parts, rns, b1, W2):
    def body(p_ref, rns_ref, b_ref, w_ref, o_ref):
        p = p_ref[...]
        rns = rns_ref[...]
        h = jnp.maximum(rns * (p[0, :N, :HIDDEN] + p[1, :N, :HIDDEN])
                        + b_ref[...], 0.0)
        o_ref[:, :HIDDEN] = jnp.dot(h, w_ref[...],
                                    preferred_element_type=jnp.float32) * rns
        o_ref[:, HIDDEN:] = jnp.zeros((N, W - HIDDEN), jnp.float32)

    return pl.pallas_call(
        body, out_shape=jax.ShapeDtypeStruct((N, W), jnp.float32),
    )(parts, rns, b1, W2)


def _head(parts, rns, b2, seg, Wd, bd, Wo, bo):
    def body(p_ref, rns_ref, b_ref, seg_ref, wd_ref, bd_ref, wo_ref, bo_ref,
             o_ref):
        p = p_ref[...]
        h2 = jnp.maximum(rns_ref[...] * (p[0, :N, :HIDDEN] + p[1, :N, :HIDDEN])
                         + b_ref[...], 0.0)
        gid = lax.broadcasted_iota(jnp.int32, (N_GRAPHS, N), 0)
        mask = jnp.where(gid == seg_ref[...], 1.0, 0.0)
        hi = h2.astype(jnp.bfloat16).astype(jnp.float32)
        lo = h2 - hi
        g = (jnp.dot(mask, hi, preferred_element_type=jnp.float32)
             + jnp.dot(mask, lo, preferred_element_type=jnp.float32))
        g = jnp.maximum(
            jnp.dot(g, wd_ref[...], preferred_element_type=jnp.float32)
            + bd_ref[...], 0.0)
        o_ref[...] = (jnp.dot(g, wo_ref[...],
                              preferred_element_type=jnp.float32)
                      + bo_ref[...])

    return pl.pallas_call(
        body, out_shape=jax.ShapeDtypeStruct((N_GRAPHS, 1), jnp.float32),
    )(parts, rns, b2, seg, Wd, bd, Wo, bo)


def kernel(x, edge_index, i, W1, b1, W2, b2, Wd, bd, Wo, bo):
    pad = NCH_ALLOC * CHUNK - EPW
    src2 = jnp.pad(edge_index[0].reshape(NW, EPW), ((0, 0), (0, pad)))
    dst3 = jnp.pad(edge_index[1].reshape(NW, EPW), ((0, 0), (0, pad)),
                   constant_values=N_PAD - 1).reshape(NW, NCH_ALLOC, CHUNK)
    ones = jnp.ones((CHUNK, W), jnp.float32)
    zeros = jnp.zeros((N_PAD, W), jnp.float32)

    degp = _deg_partial(dst3, ones, zeros)
    hs1, rns = _scale1(x, W1, degp)
    p1 = _prop_partial(hs1, src2, dst3, zeros)
    hs2 = _mid(p1, rns, b1.reshape(1, HIDDEN), W2)
    p2 = _prop_partial(hs2, src2, dst3, zeros)
    out = _head(p2, rns, b2.reshape(1, HIDDEN), i.reshape(1, N),
                Wd, bd.reshape(1, HIDDEN), Wo, bo.reshape(1, 1))
    return out


# speedup vs baseline: 3.7767x; 3.7767x over previous
import functools

import jax
import jax.numpy as jnp
from jax import lax
from jax.experimental import pallas as pl
from jax.experimental.pallas import tpu as pltpu
from jax.experimental.pallas import tpu_sc as plsc

N = 10000
E = 320000
D_FEAT = 128
HIDDEN = 64
N_GRAPHS = 64

NC = 2
NS = 16
NW = NC * NS
EPW = E // NW
CHUNK = 80
NCHUNK = EPW // CHUNK
NCH_RUN = 128
NCH_ALLOC = 130
EPW_PAD = NCH_ALLOC * CHUNK
N_PAD = 10240
RPT = N_PAD // NS
W = 128


def _sc_mesh():
    return plsc.VectorSubcoreMesh(
        core_axis_name="c", subcore_axis_name="s",
        num_cores=NC, num_subcores=NS)


def _deg_partial(dst3, ones, zeros):
    @functools.partial(
        pl.kernel,
        out_type=jax.ShapeDtypeStruct((NC, N_PAD, W), jnp.float32),
        mesh=_sc_mesh(),
        scratch_types=[
            pltpu.VMEM((NCH_ALLOC, CHUNK), jnp.int32),
            pltpu.VMEM((CHUNK, W), jnp.float32),
            pltpu.VMEM_SHARED((N_PAD, W), jnp.float32),
            pltpu.SemaphoreType.DMA,
        ],
    )
    def kern(dst_hbm, ones_hbm, zeros_hbm, out_hbm, dst_v, ones_v, acc_sh,
             dsem):
        c = lax.axis_index("c")
        s = lax.axis_index("s")
        wid = c * NS + s
        r0 = s * RPT
        pltpu.sync_copy(zeros_hbm.at[pl.ds(r0, RPT)], acc_sh.at[pl.ds(r0, RPT)])
        pltpu.sync_copy(ones_hbm, ones_v)
        pltpu.sync_copy(dst_hbm.at[wid], dst_v)
        plsc.subcore_barrier()

        def issue(j, carry):
            pltpu.async_copy(ones_v, acc_sh.at[dst_v.at[j]], dsem, add=True)
            return carry

        lax.fori_loop(0, NCHUNK, issue, 0)

        def drain(j, carry):
            pltpu.make_async_copy(ones_v, acc_sh.at[dst_v.at[j]], dsem).wait()
            return carry

        lax.fori_loop(0, NCHUNK, drain, 0)
        plsc.subcore_barrier()
        pltpu.sync_copy(acc_sh.at[pl.ds(r0, RPT)],
                        out_hbm.at[c, pl.ds(r0, RPT)])

    return kern(dst3, ones, zeros)


def _prop_partial(hs, src3, dst3, zeros):
    @functools.partial(
        pl.kernel,
        out_type=jax.ShapeDtypeStruct((NC, N_PAD, W), jnp.float32),
        mesh=_sc_mesh(),
        scratch_types=[
            pltpu.VMEM((EPW,), jnp.int32),
            pltpu.VMEM((NCH_ALLOC, CHUNK), jnp.int32),
            pltpu.VMEM((CHUNK, W), jnp.float32),
            pltpu.VMEM((CHUNK, W), jnp.float32),
            pltpu.VMEM_SHARED((N_PAD, W), jnp.float32),
            pltpu.SemaphoreType.DMA,
            pltpu.SemaphoreType.DMA,
        ],
    )
    def kern(hs_hbm, src_hbm, dst_hbm, zeros_hbm, out_hbm,
             src_v, dst_v, rows_a, rows_b, acc_sh, sem_a, sem_b):
        c = lax.axis_index("c")
        s = lax.axis_index("s")
        wid = c * NS + s
        r0 = s * RPT
        pltpu.sync_copy(zeros_hbm.at[pl.ds(r0, RPT)], acc_sh.at[pl.ds(r0, RPT)])
        pltpu.sync_copy(src_hbm.at[wid], src_v)
        pltpu.sync_copy(dst_hbm.at[wid], dst_v)
        plsc.subcore_barrier()

        def start_g(j, buf, sem):
            pltpu.async_copy(hs_hbm.at[src_v.at[pl.ds(j * CHUNK, CHUNK)]],
                             buf, sem)

        def wait_g(j, buf, sem):
            pltpu.make_async_copy(hs_hbm.at[src_v.at[pl.ds(j * CHUNK, CHUNK)]],
                                  buf, sem).wait()

        def scat(j, buf):
            pltpu.sync_copy(buf, acc_sh.at[dst_v.at[j]], add=True)

        start_g(0, rows_a, sem_a)

        def body(t, carry):
            j0 = 2 * t
            start_g(j0 + 1, rows_b, sem_b)
            wait_g(j0, rows_a, sem_a)
            scat(j0, rows_a)
            start_g(j0 + 2, rows_a, sem_a)
            wait_g(j0 + 1, rows_b, sem_b)
            scat(j0 + 1, rows_b)
            return carry

        lax.fori_loop(0, (NCHUNK - 1) // 2, body, 0)
        wait_g(NCHUNK - 1, rows_a, sem_a)
        scat(NCHUNK - 1, rows_a)
        plsc.subcore_barrier()
        pltpu.sync_copy(acc_sh.at[pl.ds(r0, RPT)],
                        out_hbm.at[c, pl.ds(r0, RPT)])

    return kern(hs, src3, dst3, zeros)


def _scale1(x, W1, degp):
    def body(x_ref, w_ref, degp_ref, hs_ref, rns_ref):
        p = degp_ref[...]
        deg = p[0, :N, 0:1] + p[1, :N, 0:1]
        rns = lax.rsqrt(jnp.maximum(deg, 1.0))
        rns_ref[...] = rns
        h = jnp.dot(x_ref[...], w_ref[...],
                    preferred_element_type=jnp.float32) * rns
        hs_ref[:, :HIDDEN] = h
        hs_ref[:, HIDDEN:] = jnp.zeros((N, W - HIDDEN), jnp.float32)

    return pl.pallas_call(
        body,
        out_shape=(jax.ShapeDtypeStruct((N, W), jnp.float32),
                   jax.ShapeDtypeStruct((N, 1), jnp.float32)),
    )(x, W1, degp)


def _mid(parts, rns, b1, W2):
    def body(p_ref, rns_ref, b_ref, w_ref, o_ref):
        p = p_ref[...]
        rns = rns_ref[...]
        h = jnp.maximum(rns * (p[0, :N, :HIDDEN] + p[1, :N, :HIDDEN])
                        + b_ref[...], 0.0)
        o_ref[:, :HIDDEN] = jnp.dot(h, w_ref[...],
                                    preferred_element_type=jnp.float32) * rns
        o_ref[:, HIDDEN:] = jnp.zeros((N, W - HIDDEN), jnp.float32)

    return pl.pallas_call(
        body, out_shape=jax.ShapeDtypeStruct((N, W), jnp.float32),
    )(parts, rns, b1, W2)


def _head(parts, rns, b2, seg, Wd, bd, Wo, bo):
    def body(p_ref, rns_ref, b_ref, seg_ref, wd_ref, bd_ref, wo_ref, bo_ref,
             o_ref):
        p = p_ref[...]
        h2 = jnp.maximum(rns_ref[...] * (p[0, :N, :HIDDEN] + p[1, :N, :HIDDEN])
                         + b_ref[...], 0.0)
        gid = lax.broadcasted_iota(jnp.int32, (N_GRAPHS, N), 0)
        mask = jnp.where(gid == seg_ref[...], 1.0, 0.0)
        hi = h2.astype(jnp.bfloat16).astype(jnp.float32)
        lo = h2 - hi
        g = (jnp.dot(mask, hi, preferred_element_type=jnp.float32)
             + jnp.dot(mask, lo, preferred_element_type=jnp.float32))
        g = jnp.maximum(
            jnp.dot(g, wd_ref[...], preferred_element_type=jnp.float32)
            + bd_ref[...], 0.0)
        o_ref[...] = (jnp.dot(g, wo_ref[...],
                              preferred_element_type=jnp.float32)
                      + bo_ref[...])

    return pl.pallas_call(
        body, out_shape=jax.ShapeDtypeStruct((N_GRAPHS, 1), jnp.float32),
    )(parts, rns, b2, seg, Wd, bd, Wo, bo)


def kernel(x, edge_index, i, W1, b1, W2, b2, Wd, bd, Wo, bo):
    pad = NCH_ALLOC * CHUNK - EPW
    src2 = edge_index[0].reshape(NW, EPW)
    dst3 = jnp.pad(edge_index[1].reshape(NW, EPW), ((0, 0), (0, pad)),
                   constant_values=N_PAD - 1).reshape(NW, NCH_ALLOC, CHUNK)
    ones = jnp.ones((CHUNK, W), jnp.float32)
    zeros = jnp.zeros((N_PAD, W), jnp.float32)

    degp = _deg_partial(dst3, ones, zeros)
    hs1, rns = _scale1(x, W1, degp)
    p1 = _prop_partial(hs1, src2, dst3, zeros)
    hs2 = _mid(p1, rns, b1.reshape(1, HIDDEN), W2)
    p2 = _prop_partial(hs2, src2, dst3, zeros)
    out = _head(p2, rns, b2.reshape(1, HIDDEN), i.reshape(1, N),
                Wd, bd.reshape(1, HIDDEN), Wo, bo.reshape(1, 1))
    return out


# speedup vs baseline: 3.7791x; 1.0006x over previous
import functools

import jax
import jax.numpy as jnp
from jax import lax
from jax.experimental import pallas as pl
from jax.experimental.pallas import tpu as pltpu
from jax.experimental.pallas import tpu_sc as plsc

N = 10000
E = 320000
D_FEAT = 128
HIDDEN = 64
N_GRAPHS = 64

NC = 2
NS = 16
NW = NC * NS
EPW = E // NW
CHUNK = 80
NCHUNK = EPW // CHUNK
NCH_ALLOC = 128
N_PAD = 10240
RPT = N_PAD // NS
W = 128


def _sc_mesh():
    return plsc.VectorSubcoreMesh(
        core_axis_name="c", subcore_axis_name="s",
        num_cores=NC, num_subcores=NS)


def _deg_partial(dst3, ones, zeros):
    @functools.partial(
        pl.kernel,
        out_type=jax.ShapeDtypeStruct((NC, N_PAD, W), jnp.float32),
        mesh=_sc_mesh(),
        scratch_types=[
            pltpu.VMEM((NCH_ALLOC, CHUNK), jnp.int32),
            pltpu.VMEM((CHUNK, W), jnp.float32),
            pltpu.VMEM_SHARED((N_PAD, W), jnp.float32),
            pltpu.SemaphoreType.DMA,
        ],
    )
    def kern(dst_hbm, ones_hbm, zeros_hbm, out_hbm, dst_v, ones_v, acc_sh,
             dsem):
        c = lax.axis_index("c")
        s = lax.axis_index("s")
        wid = c * NS + s
        r0 = s * RPT
        pltpu.sync_copy(zeros_hbm.at[pl.ds(r0, RPT)], acc_sh.at[pl.ds(r0, RPT)])
        pltpu.sync_copy(ones_hbm, ones_v)
        pltpu.sync_copy(dst_hbm.at[wid], dst_v)
        plsc.subcore_barrier()

        def issue(j, carry):
            pltpu.async_copy(ones_v, acc_sh.at[dst_v.at[j]], dsem, add=True)
            return carry

        lax.fori_loop(0, NCHUNK, issue, 0)

        def drain(j, carry):
            pltpu.make_async_copy(ones_v, acc_sh.at[dst_v.at[j]], dsem).wait()
            return carry

        lax.fori_loop(0, NCHUNK, drain, 0)
        plsc.subcore_barrier()
        pltpu.sync_copy(acc_sh.at[pl.ds(r0, RPT)],
                        out_hbm.at[c, pl.ds(r0, RPT)])

    return kern(dst3, ones, zeros)


def _prop_partial(hs, src3, dst3, zeros):
    @functools.partial(
        pl.kernel,
        out_type=jax.ShapeDtypeStruct((NC, N_PAD, W), jnp.float32),
        mesh=_sc_mesh(),
        scratch_types=[
            pltpu.VMEM((EPW,), jnp.int32),
            pltpu.VMEM((NCH_ALLOC, CHUNK), jnp.int32),
            pltpu.VMEM((CHUNK, W), jnp.float32),
            pltpu.VMEM((CHUNK, W), jnp.float32),
            pltpu.VMEM_SHARED((N_PAD, W), jnp.float32),
            pltpu.SemaphoreType.DMA,
            pltpu.SemaphoreType.DMA,
        ],
    )
    def kern(hs_hbm, src_hbm, dst_hbm, zeros_hbm, out_hbm,
             src_v, dst_v, rows_a, rows_b, acc_sh, sem_a, sem_b):
        c = lax.axis_index("c")
        s = lax.axis_index("s")
        wid = c * NS + s
        r0 = s * RPT
        pltpu.sync_copy(zeros_hbm.at[pl.ds(r0, RPT)], acc_sh.at[pl.ds(r0, RPT)])
        pltpu.sync_copy(src_hbm.at[wid], src_v)
        pltpu.sync_copy(dst_hbm.at[wid], dst_v)
        plsc.subcore_barrier()

        def start_g(j, buf, sem):
            pltpu.async_copy(hs_hbm.at[src_v.at[pl.ds(j * CHUNK, CHUNK)]],
                             buf, sem)

        def wait_g(j, buf, sem):
            pltpu.make_async_copy(hs_hbm.at[src_v.at[pl.ds(j * CHUNK, CHUNK)]],
                                  buf, sem).wait()

        def scat(j, buf):
            pltpu.sync_copy(buf, acc_sh.at[dst_v.at[j]], add=True)

        start_g(0, rows_a, sem_a)

        def body(t, carry):
            j0 = 2 * t
            start_g(j0 + 1, rows_b, sem_b)
            wait_g(j0, rows_a, sem_a)
            scat(j0, rows_a)
            start_g(j0 + 2, rows_a, sem_a)
            wait_g(j0 + 1, rows_b, sem_b)
            scat(j0 + 1, rows_b)
            return carry

        lax.fori_loop(0, (NCHUNK - 1) // 2, body, 0)
        wait_g(NCHUNK - 1, rows_a, sem_a)
        scat(NCHUNK - 1, rows_a)
        plsc.subcore_barrier()
        pltpu.sync_copy(acc_sh.at[pl.ds(r0, RPT)],
                        out_hbm.at[c, pl.ds(r0, RPT)])

    return kern(hs, src3, dst3, zeros)


def _scale1(x, W1, degp):
    def body(x_ref, w_ref, degp_ref, hs_ref, rns_ref):
        p = degp_ref[...]
        deg = p[0, :N, 0:1] + p[1, :N, 0:1]
        rns = lax.rsqrt(jnp.maximum(deg, 1.0))
        rns_ref[...] = rns
        h = jnp.dot(x_ref[...], w_ref[...],
                    preferred_element_type=jnp.float32) * rns
        hs_ref[:, :HIDDEN] = h
        hs_ref[:, HIDDEN:] = jnp.zeros((N, W - HIDDEN), jnp.float32)

    return pl.pallas_call(
        body,
        out_shape=(jax.ShapeDtypeStruct((N, W), jnp.float32),
                   jax.ShapeDtypeStruct((N, 1), jnp.float32)),
    )(x, W1, degp)


def _mid(parts, rns, b1, W2):
    def body(p_ref, rns_ref, b_ref, w_ref, o_ref):
        p = p_ref[...]
        rns = rns_ref[...]
        h = jnp.maximum(rns * (p[0, :N, :HIDDEN] + p[1, :N, :HIDDEN])
                        + b_ref[...], 0.0)
        o_ref[:, :HIDDEN] = jnp.dot(h, w_ref[...],
                                    preferred_element_type=jnp.float32) * rns
        o_ref[:, HIDDEN:] = jnp.zeros((N, W - HIDDEN), jnp.float32)

    return pl.pallas_call(
        body, out_shape=jax.ShapeDtypeStruct((N, W), jnp.float32),
    )(parts, rns, b1, W2)


def _head(parts, rns, b2, seg, Wd, bd, Wo, bo):
    def body(p_ref, rns_ref, b_ref, seg_ref, wd_ref, bd_ref, wo_ref, bo_ref,
             o_ref):
        p = p_ref[...]
        h2 = jnp.maximum(rns_ref[...] * (p[0, :N, :HIDDEN] + p[1, :N, :HIDDEN])
                         + b_ref[...], 0.0)
        gid = lax.broadcasted_iota(jnp.int32, (N_GRAPHS, N), 0)
        mask = jnp.where(gid == seg_ref[...], 1.0, 0.0)
        hi = h2.astype(jnp.bfloat16).astype(jnp.float32)
        lo = h2 - hi
        g = (jnp.dot(mask, hi, preferred_element_type=jnp.float32)
             + jnp.dot(mask, lo, preferred_element_type=jnp.float32))
        g = jnp.maximum(
            jnp.dot(g, wd_ref[...], preferred_element_type=jnp.float32)
            + bd_ref[...], 0.0)
        o_ref[...] = (jnp.dot(g, wo_ref[...],
                              preferred_element_type=jnp.float32)
                      + bo_ref[...])

    return pl.pallas_call(
        body, out_shape=jax.ShapeDtypeStruct((N_GRAPHS, 1), jnp.float32),
    )(parts, rns, b2, seg, Wd, bd, Wo, bo)


def kernel(x, edge_index, i, W1, b1, W2, b2, Wd, bd, Wo, bo):
    pad = NCH_ALLOC * CHUNK - EPW
    src2 = edge_index[0].reshape(NW, EPW)
    dst3 = jnp.pad(edge_index[1].reshape(NW, EPW), ((0, 0), (0, pad)),
                   constant_values=N_PAD - 1).reshape(NW, NCH_ALLOC, CHUNK)
    ones = jnp.ones((CHUNK, W), jnp.float32)
    zeros = jnp.zeros((N_PAD, W), jnp.float32)

    degp = _deg_partial(dst3, ones, zeros)
    hs1, rns = _scale1(x, W1, degp)
    p1 = _prop_partial(hs1, src2, dst3, zeros)
    hs2 = _mid(p1, rns, b1.reshape(1, HIDDEN), W2)
    p2 = _prop_partial(hs2, src2, dst3, zeros)
    out = _head(p2, rns, b2.reshape(1, HIDDEN), i.reshape(1, N),
                Wd, bd.reshape(1, HIDDEN), Wo, bo.reshape(1, 1))
    return out


# speedup vs baseline: 3.8515x; 1.0192x over previous
import functools

import jax
import jax.numpy as jnp
from jax import lax
from jax.experimental import pallas as pl
from jax.experimental.pallas import tpu as pltpu
from jax.experimental.pallas import tpu_sc as plsc

N = 10000
E = 320000
D_FEAT = 128
HIDDEN = 64
N_GRAPHS = 64

NC = 2
NS = 16
NW = NC * NS
EPW = E // NW
CHUNK = 80
NCHUNK = EPW // CHUNK
NCH_ALLOC = 128
N_PAD = 10240
RPT = N_PAD // NS
W = 128


def _sc_mesh():
    return plsc.VectorSubcoreMesh(
        core_axis_name="c", subcore_axis_name="s",
        num_cores=NC, num_subcores=NS)


def _deg_partial(dst3, ones, zeros):
    @functools.partial(
        pl.kernel,
        out_type=jax.ShapeDtypeStruct((NC, N_PAD, W), jnp.float32),
        mesh=_sc_mesh(),
        scratch_types=[
            pltpu.VMEM((NCH_ALLOC, CHUNK), jnp.int32),
            pltpu.VMEM((CHUNK, W), jnp.float32),
            pltpu.VMEM_SHARED((N_PAD, W), jnp.float32),
            pltpu.SemaphoreType.DMA,
        ],
    )
    def kern(dst_hbm, ones_hbm, zeros_hbm, out_hbm, dst_v, ones_v, acc_sh,
             dsem):
        c = lax.axis_index("c")
        s = lax.axis_index("s")
        wid = c * NS + s
        r0 = s * RPT
        pltpu.async_copy(zeros_hbm.at[pl.ds(r0, RPT)],
                         acc_sh.at[pl.ds(r0, RPT)], dsem)
        pltpu.sync_copy(ones_hbm, ones_v)
        pltpu.sync_copy(dst_hbm.at[wid], dst_v)
        pltpu.make_async_copy(zeros_hbm.at[pl.ds(r0, RPT)],
                              acc_sh.at[pl.ds(r0, RPT)], dsem).wait()
        plsc.subcore_barrier()

        def issue(j, carry):
            pltpu.async_copy(ones_v, acc_sh.at[dst_v.at[j]], dsem, add=True)
            return carry

        lax.fori_loop(0, NCHUNK, issue, 0)

        def drain(j, carry):
            pltpu.make_async_copy(ones_v, acc_sh.at[dst_v.at[j]], dsem).wait()
            return carry

        lax.fori_loop(0, NCHUNK, drain, 0)
        plsc.subcore_barrier()
        pltpu.sync_copy(acc_sh.at[pl.ds(r0, RPT)],
                        out_hbm.at[c, pl.ds(r0, RPT)])

    return kern(dst3, ones, zeros)


def _prop_partial(hs, src3, dst3, zeros):
    @functools.partial(
        pl.kernel,
        out_type=jax.ShapeDtypeStruct((NC, N_PAD, W), jnp.float32),
        mesh=_sc_mesh(),
        scratch_types=[
            pltpu.VMEM((EPW,), jnp.int32),
            pltpu.VMEM((NCH_ALLOC, CHUNK), jnp.int32),
            pltpu.VMEM((CHUNK, W), jnp.float32),
            pltpu.VMEM((CHUNK, W), jnp.float32),
            pltpu.VMEM_SHARED((N_PAD, W), jnp.float32),
            pltpu.SemaphoreType.DMA,
            pltpu.SemaphoreType.DMA,
        ],
    )
    def kern(hs_hbm, src_hbm, dst_hbm, zeros_hbm, out_hbm,
             src_v, dst_v, rows_a, rows_b, acc_sh, sem_a, sem_b):
        c = lax.axis_index("c")
        s = lax.axis_index("s")
        wid = c * NS + s
        r0 = s * RPT
        pltpu.async_copy(zeros_hbm.at[pl.ds(r0, RPT)],
                         acc_sh.at[pl.ds(r0, RPT)], sem_b)
        pltpu.sync_copy(src_hbm.at[wid], src_v)
        pltpu.sync_copy(dst_hbm.at[wid], dst_v)

        def start_g(j, buf, sem):
            pltpu.async_copy(hs_hbm.at[src_v.at[pl.ds(j * CHUNK, CHUNK)]],
                             buf, sem)

        def wait_g(j, buf, sem):
            pltpu.make_async_copy(hs_hbm.at[src_v.at[pl.ds(j * CHUNK, CHUNK)]],
                                  buf, sem).wait()

        def scat(j, buf):
            pltpu.sync_copy(buf, acc_sh.at[dst_v.at[j]], add=True)

        start_g(0, rows_a, sem_a)
        pltpu.make_async_copy(zeros_hbm.at[pl.ds(r0, RPT)],
                              acc_sh.at[pl.ds(r0, RPT)], sem_b).wait()
        plsc.subcore_barrier()

        def body(t, carry):
            j0 = 2 * t
            start_g(j0 + 1, rows_b, sem_b)
            wait_g(j0, rows_a, sem_a)
            scat(j0, rows_a)
            start_g(j0 + 2, rows_a, sem_a)
            wait_g(j0 + 1, rows_b, sem_b)
            scat(j0 + 1, rows_b)
            return carry

        lax.fori_loop(0, (NCHUNK - 1) // 2, body, 0)
        wait_g(NCHUNK - 1, rows_a, sem_a)
        scat(NCHUNK - 1, rows_a)
        plsc.subcore_barrier()
        pltpu.sync_copy(acc_sh.at[pl.ds(r0, RPT)],
                        out_hbm.at[c, pl.ds(r0, RPT)])

    return kern(hs, src3, dst3, zeros)


def _scale1(x, W1, degp):
    def body(x_ref, w_ref, degp_ref, hs_ref, rns_ref):
        p = degp_ref[...]
        deg = p[0, :N, 0:1] + p[1, :N, 0:1]
        rns = lax.rsqrt(jnp.maximum(deg, 1.0))
        rns_ref[...] = rns
        h = jnp.dot(x_ref[...], w_ref[...],
                    preferred_element_type=jnp.float32) * rns
        hs_ref[:, :HIDDEN] = h
        hs_ref[:, HIDDEN:] = jnp.zeros((N, W - HIDDEN), jnp.float32)

    return pl.pallas_call(
        body,
        out_shape=(jax.ShapeDtypeStruct((N, W), jnp.float32),
                   jax.ShapeDtypeStruct((N, 1), jnp.float32)),
    )(x, W1, degp)


def _mid(parts, rns, b1, W2):
    def body(p_ref, rns_ref, b_ref, w_ref, o_ref):
        p = p_ref[...]
        rns = rns_ref[...]
        h = jnp.maximum(rns * (p[0, :N, :HIDDEN] + p[1, :N, :HIDDEN])
                        + b_ref[...], 0.0)
        o_ref[:, :HIDDEN] = jnp.dot(h, w_ref[...],
                                    preferred_element_type=jnp.float32) * rns
        o_ref[:, HIDDEN:] = jnp.zeros((N, W - HIDDEN), jnp.float32)

    return pl.pallas_call(
        body, out_shape=jax.ShapeDtypeStruct((N, W), jnp.float32),
    )(parts, rns, b1, W2)


def _head(parts, rns, b2, seg, Wd, bd, Wo, bo):
    def body(p_ref, rns_ref, b_ref, seg_ref, wd_ref, bd_ref, wo_ref, bo_ref,
             o_ref):
        p = p_ref[...]
        h2 = jnp.maximum(rns_ref[...] * (p[0, :N, :HIDDEN] + p[1, :N, :HIDDEN])
                         + b_ref[...], 0.0)
        gid = lax.broadcasted_iota(jnp.int32, (N_GRAPHS, N), 0)
        mask = jnp.where(gid == seg_ref[...], 1.0, 0.0)
        hi = h2.astype(jnp.bfloat16).astype(jnp.float32)
        lo = h2 - hi
        g = (jnp.dot(mask, hi, preferred_element_type=jnp.float32)
             + jnp.dot(mask, lo, preferred_element_type=jnp.float32))
        g = jnp.maximum(
            jnp.dot(g, wd_ref[...], preferred_element_type=jnp.float32)
            + bd_ref[...], 0.0)
        o_ref[...] = (jnp.dot(g, wo_ref[...],
                              preferred_element_type=jnp.float32)
                      + bo_ref[...])

    return pl.pallas_call(
        body, out_shape=jax.ShapeDtypeStruct((N_GRAPHS, 1), jnp.float32),
    )(parts, rns, b2, seg, Wd, bd, Wo, bo)


def kernel(x, edge_index, i, W1, b1, W2, b2, Wd, bd, Wo, bo):
    pad = NCH_ALLOC * CHUNK - EPW
    src2 = edge_index[0].reshape(NW, EPW)
    dst3 = jnp.pad(edge_index[1].reshape(NW, EPW), ((0, 0), (0, pad)),
                   constant_values=N_PAD - 1).reshape(NW, NCH_ALLOC, CHUNK)
    ones = jnp.ones((CHUNK, W), jnp.float32)
    zeros = jnp.zeros((N_PAD, W), jnp.float32)

    degp = _deg_partial(dst3, ones, zeros)
    hs1, rns = _scale1(x, W1, degp)
    p1 = _prop_partial(hs1, src2, dst3, zeros)
    hs2 = _mid(p1, rns, b1.reshape(1, HIDDEN), W2)
    p2 = _prop_partial(hs2, src2, dst3, zeros)
    out = _head(p2, rns, b2.reshape(1, HIDDEN), i.reshape(1, N),
                Wd, bd.reshape(1, HIDDEN), Wo, bo.reshape(1, 1))
    return out
